# baseline (speedup 1.0000x reference)
import functools

import jax
import jax.numpy as jnp
from jax import lax
from jax.experimental import pallas as pl
from jax.experimental.pallas import tpu as pltpu
from jax.experimental.pallas import tpu_sc as plsc

MAX_TILES = 4
BATCH = 8
PATCHES = 1025
HIDDEN = 1280
LANES = 16
NCORES = 2
NSUB = 16
NWORKERS = NCORES * NSUB
ROWS_PER_CHUNK = 16
NCHUNKS = PATCHES // ROWS_PER_CHUNK
SUBROWS = HIDDEN // 128
CHUNK_128 = ROWS_PER_CHUNK * SUBROWS
TAIL_OFF = NCHUNKS * CHUNK_128
VECS_PER_128 = 128 // LANES
ZERO_ROW = MAX_TILES * MAX_TILES


def _body(hid_ref, eidx_ref, tab_ref, out_ref, idxbuf, e_buf, chunk_buf,
          tail_buf, sem):
    c_idx = lax.axis_index("c")
    s_idx = lax.axis_index("s")
    w = s_idx * NCORES + c_idx

    pltpu.sync_copy(eidx_ref.at[w], idxbuf)
    pltpu.async_copy(tab_ref.at[idxbuf], e_buf, sem).wait()

    def chunk_body(ci, carry):
        pltpu.sync_copy(hid_ref.at[w, pl.ds(ci * CHUNK_128, CHUNK_128)],
                        chunk_buf)

        def row_body(r, carry2):
            for m in range(SUBROWS):
                for k in range(VECS_PER_128):
                    j = m * VECS_PER_128 + k
                    chunk_buf[r * SUBROWS + m, pl.ds(k * LANES, LANES)] = (
                        chunk_buf[r * SUBROWS + m, pl.ds(k * LANES, LANES)]
                        + e_buf[0, pl.ds(j * LANES, LANES)])
            return carry2

        lax.fori_loop(0, ROWS_PER_CHUNK, row_body, 0)
        pltpu.sync_copy(chunk_buf,
                        out_ref.at[w, pl.ds(ci * CHUNK_128, CHUNK_128)])
        return carry

    lax.fori_loop(0, NCHUNKS, chunk_body, 0)

    pltpu.sync_copy(hid_ref.at[w, pl.ds(TAIL_OFF, SUBROWS)], tail_buf)
    for m in range(SUBROWS):
        for k in range(VECS_PER_128):
            j = m * VECS_PER_128 + k
            tail_buf[m, pl.ds(k * LANES, LANES)] = (
                tail_buf[m, pl.ds(k * LANES, LANES)]
                + e_buf[0, pl.ds(j * LANES, LANES)])
    pltpu.sync_copy(tail_buf, out_ref.at[w, pl.ds(TAIL_OFF, SUBROWS)])


@jax.jit
def _run(hidden_r, eidx, table):
    mesh = plsc.VectorSubcoreMesh(core_axis_name="c", subcore_axis_name="s")
    k = functools.partial(
        pl.kernel,
        mesh=mesh,
        out_type=jax.ShapeDtypeStruct((NWORKERS, PATCHES * SUBROWS, 128),
                                      jnp.float32),
        scratch_types=[
            pltpu.VMEM((8,), jnp.int32),
            pltpu.VMEM((8, HIDDEN), jnp.float32),
            pltpu.VMEM((CHUNK_128, 128), jnp.float32),
            pltpu.VMEM((SUBROWS, 128), jnp.float32),
            pltpu.SemaphoreType.DMA,
        ],
    )(_body)
    return k(hidden_r, eidx, table)


def kernel(hidden_state, aspect_ratios, embedding, gate):
    scale = jnp.tanh(gate)[0]
    table = embedding.astype(jnp.float32).reshape(MAX_TILES * MAX_TILES,
                                                  HIDDEN) * scale
    table = jnp.concatenate([table, jnp.zeros((1, HIDDEN), jnp.float32)], axis=0)

    h = aspect_ratios[:, 0]
    wd = aspect_ratios[:, 1]
    n = h * wd
    p = jnp.arange(MAX_TILES, dtype=jnp.int32)
    sw = jnp.maximum(wd, 1)
    row = p[None, :] // sw[:, None]
    col = p[None, :] % sw[:, None]
    eidx = jnp.where(p[None, :] < n[:, None], row * MAX_TILES + col, ZERO_ROW)
    eidx = jnp.broadcast_to(eidx.reshape(NWORKERS, 1), (NWORKERS, 8))
    eidx = eidx.astype(jnp.int32)

    hidden_r = hidden_state.reshape(NWORKERS, PATCHES * SUBROWS, 128)
    out = _run(hidden_r, eidx, table)
    return out.reshape(BATCH, MAX_TILES, PATCHES, HIDDEN)


# speedup vs baseline: 1.1526x; 1.1526x over previous
import functools

import jax
import jax.numpy as jnp
from jax import lax
from jax.experimental import pallas as pl
from jax.experimental.pallas import tpu as pltpu
from jax.experimental.pallas import tpu_sc as plsc

MAX_TILES = 4
BATCH = 8
PATCHES = 1025
HIDDEN = 1280
LANES = 16
NCORES = 2
NSUB = 16
NWORKERS = NCORES * NSUB
ROWS_PER_CHUNK = 32
NCHUNKS = PATCHES // ROWS_PER_CHUNK
SUBROWS = HIDDEN // 128
CHUNK_128 = ROWS_PER_CHUNK * SUBROWS
TAIL_OFF = NCHUNKS * CHUNK_128
VECS_PER_128 = 128 // LANES
ZERO_ROW = MAX_TILES * MAX_TILES
ROW_UNROLL = 8


def _add_embedding(buf, e_buf):
    for j in range(HIDDEN // LANES):
        m, k = divmod(j, VECS_PER_128)
        ev = e_buf[0, pl.ds(j * LANES, LANES)]

        def grp(h, carry):
            base = h * (ROW_UNROLL * SUBROWS)
            for rr in range(ROW_UNROLL):
                i = base + rr * SUBROWS + m
                buf[i, pl.ds(k * LANES, LANES)] = (
                    buf[i, pl.ds(k * LANES, LANES)] + ev)
            return carry

        lax.fori_loop(0, ROWS_PER_CHUNK // ROW_UNROLL, grp, 0)


def _body(hid_ref, eidx_ref, tab_ref, out_ref, idxbuf, e_buf, buf0, buf1,
          tail_buf, sem_e, sin0, sin1, sout0, sout1):
    c_idx = lax.axis_index("c")
    s_idx = lax.axis_index("s")
    w = s_idx * NCORES + c_idx

    bufs = (buf0, buf1)
    sins = (sin0, sin1)
    souts = (sout0, sout1)

    def copy_in(ci, b):
        return pltpu.make_async_copy(
            hid_ref.at[w, pl.ds(ci * CHUNK_128, CHUNK_128)], bufs[b], sins[b])

    def copy_out(ci, b):
        return pltpu.make_async_copy(
            bufs[b], out_ref.at[w, pl.ds(ci * CHUNK_128, CHUNK_128)], souts[b])

    pltpu.sync_copy(eidx_ref.at[w], idxbuf)
    copy_in(0, 0).start()
    copy_in(1, 1).start()
    pltpu.async_copy(tab_ref.at[idxbuf], e_buf, sem_e).wait()

    def chunk_pair(g, carry):
        c0 = g * 2
        for b in range(2):
            copy_in(c0 + b, b).wait()
            _add_embedding(bufs[b], e_buf)
            copy_out(c0 + b, b).start()
        for b in range(2):
            nci = c0 + 2 + b

            @pl.when(nci < NCHUNKS)
            def _():
                copy_out(nci - 2, b).wait()
                copy_in(nci, b).start()

        return carry

    lax.fori_loop(0, NCHUNKS // 2, chunk_pair, 0)

    pltpu.sync_copy(hid_ref.at[w, pl.ds(TAIL_OFF, SUBROWS)], tail_buf)
    for m in range(SUBROWS):
        for k in range(VECS_PER_128):
            j = m * VECS_PER_128 + k
            tail_buf[m, pl.ds(k * LANES, LANES)] = (
                tail_buf[m, pl.ds(k * LANES, LANES)]
                + e_buf[0, pl.ds(j * LANES, LANES)])
    pltpu.sync_copy(tail_buf, out_ref.at[w, pl.ds(TAIL_OFF, SUBROWS)])

    for b in range(2):
        copy_out(NCHUNKS - 2 + b, b).wait()


@jax.jit
def _run(hidden_r, eidx, table):
    mesh = plsc.VectorSubcoreMesh(core_axis_name="c", subcore_axis_name="s")
    k = functools.partial(
        pl.kernel,
        mesh=mesh,
        out_type=jax.ShapeDtypeStruct((NWORKERS, PATCHES * SUBROWS, 128),
                                      jnp.float32),
        scratch_types=[
            pltpu.VMEM((8,), jnp.int32),
            pltpu.VMEM((8, HIDDEN), jnp.float32),
            pltpu.VMEM((CHUNK_128, 128), jnp.float32),
            pltpu.VMEM((CHUNK_128, 128), jnp.float32),
            pltpu.VMEM((SUBROWS, 128), jnp.float32),
            pltpu.SemaphoreType.DMA,
            pltpu.SemaphoreType.DMA,
            pltpu.SemaphoreType.DMA,
            pltpu.SemaphoreType.DMA,
            pltpu.SemaphoreType.DMA,
        ],
    )(_body)
    return k(hidden_r, eidx, table)


def kernel(hidden_state, aspect_ratios, embedding, gate):
    scale = jnp.tanh(gate)[0]
    table = embedding.astype(jnp.float32).reshape(MAX_TILES * MAX_TILES,
                                                  HIDDEN) * scale
    table = jnp.concatenate([table, jnp.zeros((1, HIDDEN), jnp.float32)], axis=0)

    h = aspect_ratios[:, 0]
    wd = aspect_ratios[:, 1]
    n = h * wd
    p = jnp.arange(MAX_TILES, dtype=jnp.int32)
    sw = jnp.maximum(wd, 1)
    row = p[None, :] // sw[:, None]
    col = p[None, :] % sw[:, None]
    eidx = jnp.where(p[None, :] < n[:, None], row * MAX_TILES + col, ZERO_ROW)
    eidx = jnp.broadcast_to(eidx.reshape(NWORKERS, 1), (NWORKERS, 8))
    eidx = eidx.astype(jnp.int32)

    hidden_r = hidden_state.reshape(NWORKERS, PATCHES * SUBROWS, 128)
    out = _run(hidden_r, eidx, table)
    return out.reshape(BATCH, MAX_TILES, PATCHES, HIDDEN)


# speedup vs baseline: 3.7650x; 3.2666x over previous
import functools

import jax
import jax.numpy as jnp
from jax import lax
from jax.experimental import pallas as pl
from jax.experimental.pallas import tpu as pltpu
from jax.experimental.pallas import tpu_sc as plsc

MAX_TILES = 4
BATCH = 8
PATCHES = 1025
HIDDEN = 1280
LANES = 16
NCORES = 2
NSUB = 16
NWORKERS = NCORES * NSUB
ROWS_PER_CHUNK = 32
NCHUNKS = PATCHES // ROWS_PER_CHUNK
TAIL_OFF = NCHUNKS * ROWS_PER_CHUNK
VECS = HIDDEN // LANES
ZERO_ROW = MAX_TILES * MAX_TILES
ROW_UNROLL = 8


def _add_embedding(buf, e_buf):
    for j in range(VECS):
        ev = e_buf[0, pl.ds(j * LANES, LANES)]

        def grp(h, carry):
            base = h * ROW_UNROLL
            for rr in range(ROW_UNROLL):
                buf[base + rr, pl.ds(j * LANES, LANES)] = (
                    buf[base + rr, pl.ds(j * LANES, LANES)] + ev)
            return carry

        lax.fori_loop(0, ROWS_PER_CHUNK // ROW_UNROLL, grp, 0)


def _body(hid_ref, eidx_ref, tab_ref, out_ref, idxbuf, e_buf, buf0, buf1,
          tail_buf, sem_e, sin0, sin1, sout0, sout1):
    c_idx = lax.axis_index("c")
    s_idx = lax.axis_index("s")
    w = s_idx * NCORES + c_idx
    b = w // MAX_TILES
    t = w % MAX_TILES

    bufs = (buf0, buf1)
    sins = (sin0, sin1)
    souts = (sout0, sout1)

    def copy_in(ci, p):
        return pltpu.make_async_copy(
            hid_ref.at[b, t, pl.ds(ci * ROWS_PER_CHUNK, ROWS_PER_CHUNK), :],
            bufs[p], sins[p])

    def copy_out(ci, p):
        return pltpu.make_async_copy(
            bufs[p],
            out_ref.at[b, t, pl.ds(ci * ROWS_PER_CHUNK, ROWS_PER_CHUNK), :],
            souts[p])

    pltpu.sync_copy(eidx_ref.at[w], idxbuf)
    copy_in(0, 0).start()
    copy_in(1, 1).start()
    pltpu.async_copy(tab_ref.at[idxbuf], e_buf, sem_e).wait()

    def chunk_pair(g, carry):
        c0 = g * 2
        for p in range(2):
            copy_in(c0 + p, p).wait()
            _add_embedding(bufs[p], e_buf)
            copy_out(c0 + p, p).start()
        for p in range(2):
            nci = c0 + 2 + p

            @pl.when(nci < NCHUNKS)
            def _():
                copy_out(nci - 2, p).wait()
                copy_in(nci, p).start()

        return carry

    lax.fori_loop(0, NCHUNKS // 2, chunk_pair, 0)

    pltpu.sync_copy(hid_ref.at[b, t, pl.ds(TAIL_OFF, 1), :], tail_buf)
    for j in range(VECS):
        tail_buf[0, pl.ds(j * LANES, LANES)] = (
            tail_buf[0, pl.ds(j * LANES, LANES)]
            + e_buf[0, pl.ds(j * LANES, LANES)])
    pltpu.sync_copy(tail_buf, out_ref.at[b, t, pl.ds(TAIL_OFF, 1), :])

    for p in range(2):
        copy_out(NCHUNKS - 2 + p, p).wait()


@jax.jit
def _run(hidden_state, eidx, table):
    mesh = plsc.VectorSubcoreMesh(core_axis_name="c", subcore_axis_name="s")
    k = functools.partial(
        pl.kernel,
        mesh=mesh,
        out_type=jax.ShapeDtypeStruct((BATCH, MAX_TILES, PATCHES, HIDDEN),
                                      jnp.float32),
        scratch_types=[
            pltpu.VMEM((8,), jnp.int32),
            pltpu.VMEM((8, HIDDEN), jnp.float32),
            pltpu.VMEM((ROWS_PER_CHUNK, HIDDEN), jnp.float32),
            pltpu.VMEM((ROWS_PER_CHUNK, HIDDEN), jnp.float32),
            pltpu.VMEM((1, HIDDEN), jnp.float32),
            pltpu.SemaphoreType.DMA,
            pltpu.SemaphoreType.DMA,
            pltpu.SemaphoreType.DMA,
            pltpu.SemaphoreType.DMA,
            pltpu.SemaphoreType.DMA,
        ],
    )(_body)
    return k(hidden_state, eidx, table)


def kernel(hidden_state, aspect_ratios, embedding, gate):
    scale = jnp.tanh(gate)[0]
    table = embedding.astype(jnp.float32).reshape(MAX_TILES * MAX_TILES,
                                                  HIDDEN) * scale
    table = jnp.concatenate([table, jnp.zeros((1, HIDDEN), jnp.float32)], axis=0)

    h = aspect_ratios[:, 0]
    wd = aspect_ratios[:, 1]
    n = h * wd
    p = jnp.arange(MAX_TILES, dtype=jnp.int32)
    sw = jnp.maximum(wd, 1)
    row = p[None, :] // sw[:, None]
    col = p[None, :] % sw[:, None]
    eidx = jnp.where(p[None, :] < n[:, None], row * MAX_TILES + col, ZERO_ROW)
    eidx = jnp.broadcast_to(eidx.reshape(NWORKERS, 1), (NWORKERS, 8))
    eidx = eidx.astype(jnp.int32)

    return _run(hidden_state, eidx, table)


# speedup vs baseline: 5.0423x; 1.3393x over previous
import functools

import jax
import jax.numpy as jnp
from jax import lax
from jax.experimental import pallas as pl
from jax.experimental.pallas import tpu as pltpu
from jax.experimental.pallas import tpu_sc as plsc

MAX_TILES = 4
BATCH = 8
PATCHES = 1025
HIDDEN = 1280
LANES = 16
NCORES = 2
NSUB = 16
NWORKERS = NCORES * NSUB
ZERO_ROW = MAX_TILES * MAX_TILES
BLOCK_P = PATCHES


def _sc_rows_body(ar_ref, tab_ref, out_ref, arbuf, idxbuf, rows_buf, sem):
    c_idx = lax.axis_index("c")
    s_idx = lax.axis_index("s")
    w = s_idx * NCORES + c_idx

    @pl.when(w == 0)
    def _():
        pltpu.sync_copy(ar_ref, arbuf)
        for g in range(NWORKERS // LANES):
            pair = lax.iota(jnp.int32, LANES) + g * LANES
            b = pair // MAX_TILES
            t = pair % MAX_TILES
            h = plsc.load_gather(arbuf, [2 * b])
            wd = plsc.load_gather(arbuf, [2 * b + 1])
            n = h * wd
            sw = jnp.maximum(wd, 1)
            row = t // sw
            col = t - row * sw
            idx = jnp.where(t < n, row * MAX_TILES + col, ZERO_ROW)
            idxbuf[pl.ds(g * LANES, LANES)] = idx
        pltpu.async_copy(tab_ref.at[idxbuf], rows_buf, sem).wait()
        pltpu.sync_copy(rows_buf, out_ref)


@jax.jit
def _sc_rows(ar16, table):
    mesh = plsc.VectorSubcoreMesh(core_axis_name="c", subcore_axis_name="s")
    k = functools.partial(
        pl.kernel,
        mesh=mesh,
        out_type=jax.ShapeDtypeStruct((NWORKERS, HIDDEN), jnp.float32),
        scratch_types=[
            pltpu.VMEM((2 * BATCH,), jnp.int32),
            pltpu.VMEM((NWORKERS,), jnp.int32),
            pltpu.VMEM((NWORKERS, HIDDEN), jnp.float32),
            pltpu.SemaphoreType.DMA,
        ],
    )(_sc_rows_body)
    return k(ar16, table)


def _tc_add_body(hid_ref, rows_ref, out_ref):
    i = pl.program_id(0)
    r = rows_ref[pl.ds(i, 1), :]
    out_ref[...] = hid_ref[...] + r[None, None, :, :]


@jax.jit
def _tc_add(hidden_state, rows):
    grid = (NWORKERS,)
    return pl.pallas_call(
        _tc_add_body,
        grid=grid,
        in_specs=[
            pl.BlockSpec((1, 1, BLOCK_P, HIDDEN),
                         lambda i: (i // MAX_TILES, i % MAX_TILES, 0, 0)),
            pl.BlockSpec((NWORKERS, HIDDEN), lambda i: (0, 0)),
        ],
        out_specs=pl.BlockSpec((1, 1, BLOCK_P, HIDDEN),
                               lambda i: (i // MAX_TILES, i % MAX_TILES,
                                          0, 0)),
        out_shape=jax.ShapeDtypeStruct((BATCH, MAX_TILES, PATCHES, HIDDEN),
                                       jnp.float32),
        compiler_params=pltpu.CompilerParams(
            dimension_semantics=("arbitrary",)),
    )(hidden_state, rows)


def kernel(hidden_state, aspect_ratios, embedding, gate):
    scale = jnp.tanh(gate)[0]
    table = embedding.astype(jnp.float32).reshape(MAX_TILES * MAX_TILES,
                                                  HIDDEN) * scale
    table = jnp.concatenate([table, jnp.zeros((1, HIDDEN), jnp.float32)], axis=0)
    h = aspect_ratios[:, 0]
    wd = aspect_ratios[:, 1]
    n = h * wd
    p = jnp.arange(MAX_TILES, dtype=jnp.int32)
    sw = jnp.maximum(wd, 1)
    row = p[None, :] // sw[:, None]
    col = p[None, :] % sw[:, None]
    eidx = jnp.where(p[None, :] < n[:, None], row * MAX_TILES + col, ZERO_ROW)
    rows = table[eidx.reshape(NWORKERS)]
    return _tc_add(hidden_state, rows)
